# Initial kernel scaffold; baseline (speedup 1.0000x reference)
import jax
import jax.numpy as jnp
from jax.experimental import pallas as pl


def kernel(node_feature, edge_index, items_ready_to_cache, gcn_W, gcn_b, bn_gamma, bn_beta, proj_W, proj_b, item_table):
    raise NotImplementedError("write your pallas kernel here")


# speedup vs baseline: 17.6288x; 17.6288x over previous
import jax
import jax.numpy as jnp
from jax import lax
from jax.experimental import pallas as pl
from jax.experimental.pallas import tpu as pltpu
from jax.experimental.pallas import tpu_sc as plsc

N = 10000
D = 128
H = 128
E = 320000
NUM_ITEMS = 128

NC = 2
NS = 16
NW = NC * NS
EPW = E // NW
CH = 80
NCHUNK = EPW // CH
NP = 10240
RPT = NP // NS


def _mesh():
    return plsc.VectorSubcoreMesh(core_axis_name="c", subcore_axis_name="s")


def _deg_call(col, zeros_n):
    def body(col_hbm, zeros_hbm, out_hbm, cidx, ones, dbuf, dacc):
        c = lax.axis_index("c")
        s = lax.axis_index("s")
        wid = c * NS + s
        for j in range(CH // 16):
            ones[pl.ds(j * 16, 16)] = jnp.ones((16,), jnp.float32)

        @pl.when(s == 0)
        def _():
            pltpu.sync_copy(zeros_hbm, dbuf)
            pltpu.sync_copy(dbuf, dacc)

        plsc.subcore_barrier()

        def step(i, carry):
            base = wid * EPW + i * CH
            pltpu.sync_copy(col_hbm.at[pl.ds(base, CH)], cidx)
            pltpu.sync_copy(ones, dacc.at[cidx], add=True)
            return carry

        lax.fori_loop(0, NCHUNK, step, 0)
        plsc.subcore_barrier()

        @pl.when(s == 0)
        def _():
            pltpu.sync_copy(dacc, dbuf)
            pltpu.sync_copy(dbuf, out_hbm.at[pl.ds(c * N, N)])

    f = pl.kernel(
        body,
        out_type=jax.ShapeDtypeStruct((NC * N,), jnp.float32),
        mesh=_mesh(),
        scratch_types=[
            pltpu.VMEM((CH,), jnp.int32),
            pltpu.VMEM((CH,), jnp.float32),
            pltpu.VMEM((N,), jnp.float32),
            pltpu.VMEM_SHARED((N,), jnp.float32),
        ],
    )
    return f(col, zeros_n)


def _scatter_call(row, col, g, zeros_nh):
    def body(row_hbm, col_hbm, g_hbm, zeros_hbm, out_hbm, ridx, cidx, rows_buf, acc, sem):
        c = lax.axis_index("c")
        s = lax.axis_index("s")
        wid = c * NS + s
        pltpu.sync_copy(zeros_hbm, acc.at[pl.ds(s * RPT, RPT)])
        plsc.subcore_barrier()

        def step(i, carry):
            base = wid * EPW + i * CH
            pltpu.sync_copy(row_hbm.at[pl.ds(base, CH)], ridx)
            pltpu.sync_copy(col_hbm.at[pl.ds(base, CH)], cidx)
            pltpu.async_copy(g_hbm.at[ridx], rows_buf, sem).wait()
            pltpu.sync_copy(rows_buf, acc.at[cidx], add=True)
            return carry

        lax.fori_loop(0, NCHUNK, step, 0)
        plsc.subcore_barrier()
        pltpu.sync_copy(
            acc.at[pl.ds(s * RPT, RPT)],
            out_hbm.at[pl.ds(c * NP + s * RPT, RPT)],
        )

    f = pl.kernel(
        body,
        out_type=jax.ShapeDtypeStruct((NC * NP, H), jnp.float32),
        mesh=_mesh(),
        scratch_types=[
            pltpu.VMEM((CH,), jnp.int32),
            pltpu.VMEM((CH,), jnp.int32),
            pltpu.VMEM((CH, H), jnp.float32),
            pltpu.VMEM_SHARED((NP, H), jnp.float32),
            pltpu.SemaphoreType.DMA,
        ],
    )
    return f(row, col, g, zeros_nh)


def _mm_call(x, w, dp):
    def body(x_ref, w_ref, dp_ref, g_ref, dinv_ref):
        deg = dp_ref[0:N] + dp_ref[N : 2 * N] + 1.0
        dinv = lax.rsqrt(deg)
        h = jnp.dot(x_ref[...], w_ref[...], preferred_element_type=jnp.float32)
        g_ref[...] = h * dinv
        dinv_ref[...] = dinv

    return pl.pallas_call(
        body,
        out_shape=(
            jax.ShapeDtypeStruct((N, H), jnp.float32),
            jax.ShapeDtypeStruct((N, 1), jnp.float32),
        ),
    )(x, w, dp)


def _fin_call(p, g, dinv, gcn_b, bn_gamma, bn_beta, proj_W, proj_b, item_table):
    def body(p_ref, g_ref, dinv_ref, b_ref, gam_ref, bet_ref, pw_ref, pb_ref, it_ref,
             scores_ref, rsu_ref):
        s_sum = p_ref[0:N] + p_ref[NP : NP + N]
        agg = dinv_ref[...] * (s_sum + g_ref[...]) + b_ref[...]
        mean = jnp.sum(agg, axis=0, keepdims=True) * (1.0 / N)
        cen = agg - mean
        var = jnp.sum(cen * cen, axis=0, keepdims=True) * (1.0 / N)
        y0 = cen[0:1, :] * lax.rsqrt(var + 1e-5) * gam_ref[...] + bet_ref[...]
        y0 = jnp.maximum(y0, 0.0)
        rsu = (
            lax.dot_general(y0, pw_ref[...], (((1,), (1,)), ((), ())),
                            preferred_element_type=jnp.float32)
            + pb_ref[...]
        )
        rsu_ref[...] = rsu
        scores_ref[...] = lax.dot_general(rsu, it_ref[...], (((1,), (1,)), ((), ())),
                                          preferred_element_type=jnp.float32)

    return pl.pallas_call(
        body,
        out_shape=(
            jax.ShapeDtypeStruct((1, NUM_ITEMS), jnp.float32),
            jax.ShapeDtypeStruct((1, D), jnp.float32),
        ),
    )(p, g, dinv, gcn_b, bn_gamma, bn_beta, proj_W, proj_b, item_table)


def kernel(node_feature, edge_index, items_ready_to_cache, gcn_W, gcn_b,
           bn_gamma, bn_beta, proj_W, proj_b, item_table):
    row = edge_index[0]
    col = edge_index[1]
    zeros_n = jnp.zeros((N,), jnp.float32)
    zeros_nh = jnp.zeros((RPT, H), jnp.float32)
    dp = _deg_call(col, zeros_n)
    g, dinv = _mm_call(node_feature, gcn_W, dp.reshape(2 * N, 1))
    p = _scatter_call(row, col, g, zeros_nh)
    scores2, rsu = _fin_call(
        p, g, dinv,
        gcn_b.reshape(1, H), bn_gamma.reshape(1, H), bn_beta.reshape(1, H),
        proj_W, proj_b.reshape(1, D), item_table,
    )
    return scores2.reshape(NUM_ITEMS), rsu


# speedup vs baseline: 38.3233x; 2.1739x over previous
import jax
import jax.numpy as jnp
from jax import lax
from jax.experimental import pallas as pl
from jax.experimental.pallas import tpu as pltpu
from jax.experimental.pallas import tpu_sc as plsc

N = 10000
D = 128
H = 128
E = 320000
NUM_ITEMS = 128

NC = 2
NS = 16
NW = NC * NS
EPW = E // NW
CH = 80
NCHUNK = EPW // CH
NP = 10240
RPT = NP // NS


def _mesh():
    return plsc.VectorSubcoreMesh(core_axis_name="c", subcore_axis_name="s")


def _deg_call(col2, zeros_n):
    def body(col_hbm, zeros_hbm, out_hbm, cidx_all, ones, dbuf, dacc):
        c = lax.axis_index("c")
        s = lax.axis_index("s")
        wid = c * NS + s
        for j in range(CH // 16):
            ones[pl.ds(j * 16, 16)] = jnp.ones((16,), jnp.float32)
        pltpu.sync_copy(col_hbm.at[wid], cidx_all)

        @pl.when(s == 0)
        def _():
            pltpu.sync_copy(zeros_hbm, dbuf)
            pltpu.sync_copy(dbuf, dacc)

        plsc.subcore_barrier()

        def step(i, carry):
            pltpu.sync_copy(ones, dacc.at[cidx_all.at[i]], add=True)
            return carry

        lax.fori_loop(0, NCHUNK, step, 0)
        plsc.subcore_barrier()

        @pl.when(s == 0)
        def _():
            pltpu.sync_copy(dacc, dbuf)
            pltpu.sync_copy(dbuf, out_hbm.at[pl.ds(c * N, N)])

    f = pl.kernel(
        body,
        out_type=jax.ShapeDtypeStruct((NC * N,), jnp.float32),
        mesh=_mesh(),
        scratch_types=[
            pltpu.VMEM((NCHUNK, CH), jnp.int32),
            pltpu.VMEM((CH,), jnp.float32),
            pltpu.VMEM((N,), jnp.float32),
            pltpu.VMEM_SHARED((N,), jnp.float32),
        ],
    )
    return f(col2, zeros_n)


def _scatter_call(row2, col2, g, zeros_nh):
    def body(row_hbm, col_hbm, g_hbm, zeros_hbm, out_hbm,
             ridx_all, cidx_all, rbuf0, rbuf1, acc, sem0, sem1):
        c = lax.axis_index("c")
        s = lax.axis_index("s")
        wid = c * NS + s
        pltpu.sync_copy(zeros_hbm, acc.at[pl.ds(s * RPT, RPT)])
        pltpu.sync_copy(row_hbm.at[pl.ds(wid * EPW, EPW)], ridx_all)
        pltpu.sync_copy(col_hbm.at[wid], cidx_all)
        plsc.subcore_barrier()

        bufs = (rbuf0, rbuf1)
        sems = (sem0, sem1)

        def gidx(i):
            return ridx_all.at[pl.ds(i * CH, CH)]

        pltpu.async_copy(g_hbm.at[gidx(0)], rbuf0, sem0)
        pltpu.async_copy(g_hbm.at[gidx(1)], rbuf1, sem1)

        npair = (NCHUNK - 1) // 2

        def step(j, carry):
            i0 = 2 * j
            for b in range(2):
                i = i0 + b
                pltpu.make_async_copy(g_hbm.at[gidx(i)], bufs[b], sems[b]).wait()
                pltpu.sync_copy(bufs[b], acc.at[cidx_all.at[i]], add=True)

                @pl.when(i + 2 < NCHUNK)
                def _():
                    pltpu.async_copy(g_hbm.at[gidx(i + 2)], bufs[b], sems[b])
            return carry

        lax.fori_loop(0, npair, step, 0)
        i = NCHUNK - 1
        pltpu.make_async_copy(g_hbm.at[gidx(i)], rbuf0, sem0).wait()
        pltpu.sync_copy(rbuf0, acc.at[cidx_all.at[i]], add=True)

        plsc.subcore_barrier()
        pltpu.sync_copy(
            acc.at[pl.ds(s * RPT, RPT)],
            out_hbm.at[pl.ds(c * NP + s * RPT, RPT)],
        )

    f = pl.kernel(
        body,
        out_type=jax.ShapeDtypeStruct((NC * NP, H), jnp.float32),
        mesh=_mesh(),
        scratch_types=[
            pltpu.VMEM((EPW,), jnp.int32),
            pltpu.VMEM((NCHUNK, CH), jnp.int32),
            pltpu.VMEM((CH, H), jnp.float32),
            pltpu.VMEM((CH, H), jnp.float32),
            pltpu.VMEM_SHARED((NP, H), jnp.float32),
            pltpu.SemaphoreType.DMA,
            pltpu.SemaphoreType.DMA,
        ],
    )
    return f(row2, col2, g, zeros_nh)


def _mm_call(x, w, dp):
    def body(x_ref, w_ref, dp_ref, g_ref, dinv_ref):
        deg = dp_ref[0:N] + dp_ref[N : 2 * N] + 1.0
        dinv = lax.rsqrt(deg)
        h = jnp.dot(x_ref[...], w_ref[...], preferred_element_type=jnp.float32)
        g_ref[...] = h * dinv
        dinv_ref[...] = dinv

    return pl.pallas_call(
        body,
        out_shape=(
            jax.ShapeDtypeStruct((N, H), jnp.float32),
            jax.ShapeDtypeStruct((N, 1), jnp.float32),
        ),
    )(x, w, dp)


def _fin_call(p, g, dinv, gcn_b, bn_gamma, bn_beta, proj_W, proj_b, item_table):
    def body(p_ref, g_ref, dinv_ref, b_ref, gam_ref, bet_ref, pw_ref, pb_ref, it_ref,
             scores_ref, rsu_ref):
        s_sum = p_ref[0:N] + p_ref[NP : NP + N]
        agg = dinv_ref[...] * (s_sum + g_ref[...]) + b_ref[...]
        mean = jnp.sum(agg, axis=0, keepdims=True) * (1.0 / N)
        cen = agg - mean
        var = jnp.sum(cen * cen, axis=0, keepdims=True) * (1.0 / N)
        y0 = cen[0:1, :] * lax.rsqrt(var + 1e-5) * gam_ref[...] + bet_ref[...]
        y0 = jnp.maximum(y0, 0.0)
        rsu = (
            lax.dot_general(y0, pw_ref[...], (((1,), (1,)), ((), ())),
                            preferred_element_type=jnp.float32)
            + pb_ref[...]
        )
        rsu_ref[...] = rsu
        scores_ref[...] = lax.dot_general(rsu, it_ref[...], (((1,), (1,)), ((), ())),
                                          preferred_element_type=jnp.float32)

    return pl.pallas_call(
        body,
        out_shape=(
            jax.ShapeDtypeStruct((1, NUM_ITEMS), jnp.float32),
            jax.ShapeDtypeStruct((1, D), jnp.float32),
        ),
    )(p, g, dinv, gcn_b, bn_gamma, bn_beta, proj_W, proj_b, item_table)


def kernel(node_feature, edge_index, items_ready_to_cache, gcn_W, gcn_b,
           bn_gamma, bn_beta, proj_W, proj_b, item_table):
    row1 = edge_index[0]
    col2 = edge_index[1].reshape(NW, NCHUNK, CH)
    zeros_n = jnp.zeros((N,), jnp.float32)
    zeros_nh = jnp.zeros((RPT, H), jnp.float32)
    dp = _deg_call(col2, zeros_n)
    g, dinv = _mm_call(node_feature, gcn_W, dp.reshape(2 * N, 1))
    p = _scatter_call(row1, col2, g, zeros_nh)
    scores2, rsu = _fin_call(
        p, g, dinv,
        gcn_b.reshape(1, H), bn_gamma.reshape(1, H), bn_beta.reshape(1, H),
        proj_W, proj_b.reshape(1, D), item_table,
    )
    return scores2.reshape(NUM_ITEMS), rsu


# speedup vs baseline: 42.3575x; 1.1053x over previous
import jax
import jax.numpy as jnp
from jax import lax
from jax.experimental import pallas as pl
from jax.experimental.pallas import tpu as pltpu
from jax.experimental.pallas import tpu_sc as plsc

N = 10000
D = 128
H = 128
E = 320000
NUM_ITEMS = 128

NC = 2
NS = 16
NW = NC * NS
EPW = E // NW
CH = 80
NCHUNK = EPW // CH
NP = 10240
RPT = NP // NS


def _mesh():
    return plsc.VectorSubcoreMesh(core_axis_name="c", subcore_axis_name="s")


def _deg_call(col2, zeros_n):
    def body(col_hbm, zeros_hbm, out_hbm, cidx_all, ones, dbuf, dacc):
        c = lax.axis_index("c")
        s = lax.axis_index("s")
        wid = c * NS + s
        for j in range(CH // 16):
            ones[pl.ds(j * 16, 16)] = jnp.ones((16,), jnp.float32)
        pltpu.sync_copy(col_hbm.at[wid], cidx_all)

        @pl.when(s == 0)
        def _():
            pltpu.sync_copy(zeros_hbm, dbuf)
            pltpu.sync_copy(dbuf, dacc)

        plsc.subcore_barrier()

        def step(i, carry):
            pltpu.sync_copy(ones, dacc.at[cidx_all.at[i]], add=True)
            return carry

        lax.fori_loop(0, NCHUNK, step, 0)
        plsc.subcore_barrier()

        @pl.when(s == 0)
        def _():
            pltpu.sync_copy(dacc, dbuf)
            pltpu.sync_copy(dbuf, out_hbm.at[pl.ds(c * N, N)])

    f = pl.kernel(
        body,
        out_type=jax.ShapeDtypeStruct((NC * N,), jnp.float32),
        mesh=_mesh(),
        scratch_types=[
            pltpu.VMEM((NCHUNK, CH), jnp.int32),
            pltpu.VMEM((CH,), jnp.float32),
            pltpu.VMEM((N,), jnp.float32),
            pltpu.VMEM_SHARED((N,), jnp.float32),
        ],
    )
    return f(col2, zeros_n)


def _scatter_call(row1, col1, g, zeros_nh):
    def body(row_hbm, col_hbm, g_hbm, zeros_hbm, out_hbm,
             ridx_all, cidx0, cidx1, cidx2, rbuf0, rbuf1, rbuf2, acc,
             gsem0, gsem1, gsem2, ssem0, ssem1, ssem2):
        c = lax.axis_index("c")
        s = lax.axis_index("s")
        wid = c * NS + s
        pltpu.sync_copy(zeros_hbm, acc.at[pl.ds(s * RPT, RPT)])
        pltpu.sync_copy(row_hbm.at[pl.ds(wid * EPW, EPW)], ridx_all)
        plsc.subcore_barrier()

        rbufs = (rbuf0, rbuf1, rbuf2)
        cidxs = (cidx0, cidx1, cidx2)
        gsems = (gsem0, gsem1, gsem2)
        ssems = (ssem0, ssem1, ssem2)

        def gsrc(i):
            return g_hbm.at[ridx_all.at[pl.ds(i * CH, CH)]]

        def csrc(i):
            return col_hbm.at[pl.ds(wid * EPW + i * CH, CH)]

        def fire(i, b):
            pltpu.async_copy(csrc(i), cidxs[b], gsems[b])
            pltpu.async_copy(gsrc(i), rbufs[b], gsems[b])

        def wait_gather(i, b):
            pltpu.make_async_copy(csrc(i), cidxs[b], gsems[b]).wait()
            pltpu.make_async_copy(gsrc(i), rbufs[b], gsems[b]).wait()

        def drain_scatter(b):
            pltpu.make_async_copy(g_hbm.at[pl.ds(0, CH)], rbufs[b], ssems[b]).wait()

        fire(0, 0)
        fire(1, 1)

        def handle(i, b, b2, prefetch, first):
            wait_gather(i, b)
            pltpu.async_copy(rbufs[b], acc.at[cidxs[b]], ssems[b], add=True)
            if prefetch:
                if first:
                    fire(i + 2, b2)
                else:
                    drain_scatter(b2)
                    fire(i + 2, b2)

        def step(j, carry):
            i0 = 3 * j

            @pl.when(j == 0)
            def _():
                for b in range(3):
                    handle(b, b, (b + 2) % 3, True, b == 0)

            @pl.when(j > 0)
            def _():
                for b in range(3):
                    i = i0 + b
                    handle(i, b, (b + 2) % 3, True, False)
            return carry

        ngroup = NCHUNK // 3
        lax.fori_loop(0, ngroup, step, 0)
        for i, b in ((NCHUNK - 2, 0), (NCHUNK - 1, 1)):
            wait_gather(i, b)
            pltpu.async_copy(rbufs[b], acc.at[cidxs[b]], ssems[b], add=True)
        for b in range(3):
            drain_scatter(b)

        plsc.subcore_barrier()
        pltpu.sync_copy(
            acc.at[pl.ds(s * RPT, RPT)],
            out_hbm.at[pl.ds(c * NP + s * RPT, RPT)],
        )

    f = pl.kernel(
        body,
        out_type=jax.ShapeDtypeStruct((NC * NP, H), jnp.float32),
        mesh=_mesh(),
        scratch_types=[
            pltpu.VMEM((EPW,), jnp.int32),
            pltpu.VMEM((CH,), jnp.int32),
            pltpu.VMEM((CH,), jnp.int32),
            pltpu.VMEM((CH,), jnp.int32),
            pltpu.VMEM((CH, H), jnp.float32),
            pltpu.VMEM((CH, H), jnp.float32),
            pltpu.VMEM((CH, H), jnp.float32),
            pltpu.VMEM_SHARED((NP, H), jnp.float32),
            pltpu.SemaphoreType.DMA,
            pltpu.SemaphoreType.DMA,
            pltpu.SemaphoreType.DMA,
            pltpu.SemaphoreType.DMA,
            pltpu.SemaphoreType.DMA,
            pltpu.SemaphoreType.DMA,
        ],
    )
    return f(row1, col1, g, zeros_nh)


def _mm_call(x, w, dp):
    def body(x_ref, w_ref, dp_ref, g_ref, dinv_ref):
        deg = dp_ref[0:N] + dp_ref[N : 2 * N] + 1.0
        dinv = lax.rsqrt(deg)
        h = jnp.dot(x_ref[...], w_ref[...], preferred_element_type=jnp.float32)
        g_ref[...] = h * dinv
        dinv_ref[...] = dinv

    return pl.pallas_call(
        body,
        out_shape=(
            jax.ShapeDtypeStruct((N, H), jnp.float32),
            jax.ShapeDtypeStruct((N, 1), jnp.float32),
        ),
    )(x, w, dp)


def _fin_call(p, g, dinv, gcn_b, bn_gamma, bn_beta, proj_W, proj_b, item_table):
    def body(p_ref, g_ref, dinv_ref, b_ref, gam_ref, bet_ref, pw_ref, pb_ref, it_ref,
             scores_ref, rsu_ref):
        s_sum = p_ref[0:N] + p_ref[NP : NP + N]
        agg = dinv_ref[...] * (s_sum + g_ref[...]) + b_ref[...]
        mean = jnp.sum(agg, axis=0, keepdims=True) * (1.0 / N)
        cen = agg - mean
        var = jnp.sum(cen * cen, axis=0, keepdims=True) * (1.0 / N)
        y0 = cen[0:1, :] * lax.rsqrt(var + 1e-5) * gam_ref[...] + bet_ref[...]
        y0 = jnp.maximum(y0, 0.0)
        rsu = (
            lax.dot_general(y0, pw_ref[...], (((1,), (1,)), ((), ())),
                            preferred_element_type=jnp.float32)
            + pb_ref[...]
        )
        rsu_ref[...] = rsu
        scores_ref[...] = lax.dot_general(rsu, it_ref[...], (((1,), (1,)), ((), ())),
                                          preferred_element_type=jnp.float32)

    return pl.pallas_call(
        body,
        out_shape=(
            jax.ShapeDtypeStruct((1, NUM_ITEMS), jnp.float32),
            jax.ShapeDtypeStruct((1, D), jnp.float32),
        ),
    )(p, g, dinv, gcn_b, bn_gamma, bn_beta, proj_W, proj_b, item_table)


def kernel(node_feature, edge_index, items_ready_to_cache, gcn_W, gcn_b,
           bn_gamma, bn_beta, proj_W, proj_b, item_table):
    row1 = edge_index[0]
    col1 = edge_index[1]
    col2 = col1.reshape(NW, NCHUNK, CH)
    zeros_n = jnp.zeros((N,), jnp.float32)
    zeros_nh = jnp.zeros((RPT, H), jnp.float32)
    dp = _deg_call(col2, zeros_n)
    g, dinv = _mm_call(node_feature, gcn_W, dp.reshape(2 * N, 1))
    p = _scatter_call(row1, col1, g, zeros_nh)
    scores2, rsu = _fin_call(
        p, g, dinv,
        gcn_b.reshape(1, H), bn_gamma.reshape(1, H), bn_beta.reshape(1, H),
        proj_W, proj_b.reshape(1, D), item_table,
    )
    return scores2.reshape(NUM_ITEMS), rsu


# speedup vs baseline: 44.0161x; 1.0392x over previous
import jax
import jax.numpy as jnp
from jax import lax
from jax.experimental import pallas as pl
from jax.experimental.pallas import tpu as pltpu
from jax.experimental.pallas import tpu_sc as plsc

N = 10000
D = 128
H = 128
E = 320000
NUM_ITEMS = 128

NC = 2
NS = 16
NW = NC * NS
EPW = E // NW
CH = 80
NCHUNK = EPW // CH
NP = 10240
RPT = NP // NS


def _mesh():
    return plsc.VectorSubcoreMesh(core_axis_name="c", subcore_axis_name="s")


def _deg_call(col2, zeros_n):
    WIN = 8

    def body(col_hbm, zeros_hbm, out_hbm, cidx_all, ones, dbuf, dacc, sem):
        c = lax.axis_index("c")
        s = lax.axis_index("s")
        wid = c * NS + s
        for j in range(CH // 16):
            ones[pl.ds(j * 16, 16)] = jnp.ones((16,), jnp.float32)
        pltpu.sync_copy(col_hbm.at[wid], cidx_all)

        @pl.when(s == 0)
        def _():
            pltpu.sync_copy(zeros_hbm, dbuf)
            pltpu.sync_copy(dbuf, dacc)

        plsc.subcore_barrier()

        def fire(i):
            pltpu.async_copy(ones, dacc.at[cidx_all.at[i]], sem, add=True)

        def drain_one():
            pltpu.make_async_copy(ones, dacc.at[cidx_all.at[0]], sem).wait()

        for i in range(WIN):
            fire(i)

        def step(i, carry):
            drain_one()
            fire(i + WIN)
            return carry

        lax.fori_loop(0, NCHUNK - WIN, step, 0)
        for _ in range(WIN):
            drain_one()
        plsc.subcore_barrier()

        @pl.when(s == 0)
        def _():
            pltpu.sync_copy(dacc, dbuf)
            pltpu.sync_copy(dbuf, out_hbm.at[pl.ds(c * N, N)])

    f = pl.kernel(
        body,
        out_type=jax.ShapeDtypeStruct((NC * N,), jnp.float32),
        mesh=_mesh(),
        scratch_types=[
            pltpu.VMEM((NCHUNK, CH), jnp.int32),
            pltpu.VMEM((CH,), jnp.float32),
            pltpu.VMEM((N,), jnp.float32),
            pltpu.VMEM_SHARED((N,), jnp.float32),
            pltpu.SemaphoreType.DMA,
        ],
    )
    return f(col2, zeros_n)


def _scatter_call(row1, col1, g, zeros_nh):
    def body(row_hbm, col_hbm, g_hbm, zeros_hbm, out_hbm,
             ridx_all, cidx0, cidx1, cidx2, rbuf0, rbuf1, rbuf2, acc,
             gsem0, gsem1, gsem2, ssem0, ssem1, ssem2):
        c = lax.axis_index("c")
        s = lax.axis_index("s")
        wid = c * NS + s
        pltpu.sync_copy(zeros_hbm, acc.at[pl.ds(s * RPT, RPT)])
        pltpu.sync_copy(row_hbm.at[pl.ds(wid * EPW, EPW)], ridx_all)
        plsc.subcore_barrier()

        rbufs = (rbuf0, rbuf1, rbuf2)
        cidxs = (cidx0, cidx1, cidx2)
        gsems = (gsem0, gsem1, gsem2)
        ssems = (ssem0, ssem1, ssem2)

        def gsrc(i):
            return g_hbm.at[ridx_all.at[pl.ds(i * CH, CH)]]

        def csrc(i):
            return col_hbm.at[pl.ds(wid * EPW + i * CH, CH)]

        def fire(i, b):
            pltpu.async_copy(csrc(i), cidxs[b], gsems[b])
            pltpu.async_copy(gsrc(i), rbufs[b], gsems[b])

        def wait_gather(i, b):
            pltpu.make_async_copy(csrc(i), cidxs[b], gsems[b]).wait()
            pltpu.make_async_copy(gsrc(i), rbufs[b], gsems[b]).wait()

        def drain_scatter(b):
            pltpu.make_async_copy(g_hbm.at[pl.ds(0, CH)], rbufs[b], ssems[b]).wait()

        fire(0, 0)
        fire(1, 1)

        def handle(i, b, b2, prefetch, first):
            wait_gather(i, b)
            pltpu.async_copy(rbufs[b], acc.at[cidxs[b]], ssems[b], add=True)
            if prefetch:
                if first:
                    fire(i + 2, b2)
                else:
                    drain_scatter(b2)
                    fire(i + 2, b2)

        def step(j, carry):
            i0 = 3 * j

            @pl.when(j == 0)
            def _():
                for b in range(3):
                    handle(b, b, (b + 2) % 3, True, b == 0)

            @pl.when(j > 0)
            def _():
                for b in range(3):
                    i = i0 + b
                    handle(i, b, (b + 2) % 3, True, False)
            return carry

        ngroup = NCHUNK // 3
        lax.fori_loop(0, ngroup, step, 0)
        for i, b in ((NCHUNK - 2, 0), (NCHUNK - 1, 1)):
            wait_gather(i, b)
            pltpu.async_copy(rbufs[b], acc.at[cidxs[b]], ssems[b], add=True)
        for b in range(3):
            drain_scatter(b)

        plsc.subcore_barrier()
        pltpu.sync_copy(
            acc.at[pl.ds(s * RPT, RPT)],
            out_hbm.at[pl.ds(c * NP + s * RPT, RPT)],
        )

    f = pl.kernel(
        body,
        out_type=jax.ShapeDtypeStruct((NC * NP, H), jnp.float32),
        mesh=_mesh(),
        scratch_types=[
            pltpu.VMEM((EPW,), jnp.int32),
            pltpu.VMEM((CH,), jnp.int32),
            pltpu.VMEM((CH,), jnp.int32),
            pltpu.VMEM((CH,), jnp.int32),
            pltpu.VMEM((CH, H), jnp.float32),
            pltpu.VMEM((CH, H), jnp.float32),
            pltpu.VMEM((CH, H), jnp.float32),
            pltpu.VMEM_SHARED((NP, H), jnp.float32),
            pltpu.SemaphoreType.DMA,
            pltpu.SemaphoreType.DMA,
            pltpu.SemaphoreType.DMA,
            pltpu.SemaphoreType.DMA,
            pltpu.SemaphoreType.DMA,
            pltpu.SemaphoreType.DMA,
        ],
    )
    return f(row1, col1, g, zeros_nh)


def _h_call(x, w):
    def body(x_ref, w_ref, h_ref):
        h_ref[...] = jnp.dot(x_ref[...], w_ref[...],
                             preferred_element_type=jnp.float32)

    return pl.pallas_call(
        body,
        out_shape=jax.ShapeDtypeStruct((N, H), jnp.float32),
    )(x, w)


def _scale_call(h, dp):
    def body(h_ref, dp_ref, g_ref, dinv_ref):
        deg = dp_ref[0:N] + dp_ref[N : 2 * N] + 1.0
        dinv = lax.rsqrt(deg)
        g_ref[...] = h_ref[...] * dinv
        dinv_ref[...] = dinv

    return pl.pallas_call(
        body,
        out_shape=(
            jax.ShapeDtypeStruct((N, H), jnp.float32),
            jax.ShapeDtypeStruct((N, 1), jnp.float32),
        ),
    )(h, dp)


def _fin_call(p, g, dinv, gcn_b, bn_gamma, bn_beta, proj_W, proj_b, item_table):
    def body(p_ref, g_ref, dinv_ref, b_ref, gam_ref, bet_ref, pw_ref, pb_ref, it_ref,
             scores_ref, rsu_ref):
        s_sum = p_ref[0:N] + p_ref[NP : NP + N]
        agg = dinv_ref[...] * (s_sum + g_ref[...]) + b_ref[...]
        mean = jnp.sum(agg, axis=0, keepdims=True) * (1.0 / N)
        cen = agg - mean
        var = jnp.sum(cen * cen, axis=0, keepdims=True) * (1.0 / N)
        y0 = cen[0:1, :] * lax.rsqrt(var + 1e-5) * gam_ref[...] + bet_ref[...]
        y0 = jnp.maximum(y0, 0.0)
        rsu = (
            lax.dot_general(y0, pw_ref[...], (((1,), (1,)), ((), ())),
                            preferred_element_type=jnp.float32)
            + pb_ref[...]
        )
        rsu_ref[...] = rsu
        scores_ref[...] = lax.dot_general(rsu, it_ref[...], (((1,), (1,)), ((), ())),
                                          preferred_element_type=jnp.float32)

    return pl.pallas_call(
        body,
        out_shape=(
            jax.ShapeDtypeStruct((1, NUM_ITEMS), jnp.float32),
            jax.ShapeDtypeStruct((1, D), jnp.float32),
        ),
    )(p, g, dinv, gcn_b, bn_gamma, bn_beta, proj_W, proj_b, item_table)


def kernel(node_feature, edge_index, items_ready_to_cache, gcn_W, gcn_b,
           bn_gamma, bn_beta, proj_W, proj_b, item_table):
    row1 = edge_index[0]
    col1 = edge_index[1]
    col2 = col1.reshape(NW, NCHUNK, CH)
    zeros_n = jnp.zeros((N,), jnp.float32)
    zeros_nh = jnp.zeros((RPT, H), jnp.float32)
    dp = _deg_call(col2, zeros_n)
    hmat = _h_call(node_feature, gcn_W)
    g, dinv = _scale_call(hmat, dp.reshape(2 * N, 1))
    p = _scatter_call(row1, col1, g, zeros_nh)
    scores2, rsu = _fin_call(
        p, g, dinv,
        gcn_b.reshape(1, H), bn_gamma.reshape(1, H), bn_beta.reshape(1, H),
        proj_W, proj_b.reshape(1, D), item_table,
    )
    return scores2.reshape(NUM_ITEMS), rsu


# speedup vs baseline: 44.1820x; 1.0038x over previous
import jax
import jax.numpy as jnp
from jax import lax
from jax.experimental import pallas as pl
from jax.experimental.pallas import tpu as pltpu
from jax.experimental.pallas import tpu_sc as plsc

N = 10000
D = 128
H = 128
E = 320000
NUM_ITEMS = 128

NC = 2
NS = 16
NW = NC * NS
EPW = E // NW
CH = 80
NCHUNK = EPW // CH
NP = 10240
RPT = NP // NS


def _mesh():
    return plsc.VectorSubcoreMesh(core_axis_name="c", subcore_axis_name="s")


def _deg_call(col2, zeros_n):
    WIN = 8

    def body(col_hbm, zeros_hbm, out_hbm, cidx_all, ones, dbuf, dacc, sem):
        c = lax.axis_index("c")
        s = lax.axis_index("s")
        wid = c * NS + s
        for j in range(CH // 16):
            ones[pl.ds(j * 16, 16)] = jnp.ones((16,), jnp.float32)
        pltpu.sync_copy(col_hbm.at[wid], cidx_all)

        @pl.when(s == 0)
        def _():
            pltpu.sync_copy(zeros_hbm, dbuf)
            pltpu.sync_copy(dbuf, dacc)

        plsc.subcore_barrier()

        def fire(i):
            pltpu.async_copy(ones, dacc.at[cidx_all.at[i]], sem, add=True)

        def drain_one():
            pltpu.make_async_copy(ones, dacc.at[cidx_all.at[0]], sem).wait()

        for i in range(WIN):
            fire(i)

        def step(i, carry):
            drain_one()
            fire(i + WIN)
            return carry

        lax.fori_loop(0, NCHUNK - WIN, step, 0)
        for _ in range(WIN):
            drain_one()
        plsc.subcore_barrier()

        @pl.when(s == 0)
        def _():
            pltpu.sync_copy(dacc, dbuf)
            pltpu.sync_copy(dbuf, out_hbm.at[pl.ds(c * N, N)])

    f = pl.kernel(
        body,
        out_type=jax.ShapeDtypeStruct((NC * N,), jnp.float32),
        mesh=_mesh(),
        scratch_types=[
            pltpu.VMEM((NCHUNK, CH), jnp.int32),
            pltpu.VMEM((CH,), jnp.float32),
            pltpu.VMEM((N,), jnp.float32),
            pltpu.VMEM_SHARED((N,), jnp.float32),
            pltpu.SemaphoreType.DMA,
        ],
    )
    return f(col2, zeros_n)


def _scatter_call(row1, col1, g, zeros_nh):
    def body(row_hbm, col_hbm, g_hbm, zeros_hbm, out_hbm,
             ridx_all, cidx0, cidx1, cidx2, rbuf0, rbuf1, rbuf2, acc,
             gsem0, gsem1, gsem2, ssem0, ssem1, ssem2):
        c = lax.axis_index("c")
        s = lax.axis_index("s")
        wid = c * NS + s
        pltpu.sync_copy(zeros_hbm, acc.at[pl.ds(s * RPT, RPT)])
        pltpu.sync_copy(row_hbm.at[pl.ds(wid * EPW, EPW)], ridx_all)
        plsc.subcore_barrier()

        rbufs = (rbuf0, rbuf1, rbuf2)
        cidxs = (cidx0, cidx1, cidx2)
        gsems = (gsem0, gsem1, gsem2)
        ssems = (ssem0, ssem1, ssem2)

        def gsrc(i):
            return g_hbm.at[ridx_all.at[pl.ds(i * CH, CH)]]

        def csrc(i):
            return col_hbm.at[pl.ds(wid * EPW + i * CH, CH)]

        def fire(i, b):
            pltpu.async_copy(csrc(i), cidxs[b], gsems[b])
            pltpu.async_copy(gsrc(i), rbufs[b], gsems[b])

        def wait_gather(i, b):
            pltpu.make_async_copy(csrc(i), cidxs[b], gsems[b]).wait()
            pltpu.make_async_copy(gsrc(i), rbufs[b], gsems[b]).wait()

        def drain_scatter(b):
            pltpu.make_async_copy(g_hbm.at[pl.ds(0, CH)], rbufs[b], ssems[b]).wait()

        fire(0, 0)
        fire(1, 1)

        def handle(i, b, b2, prefetch, first):
            wait_gather(i, b)
            pltpu.async_copy(rbufs[b], acc.at[cidxs[b]], ssems[b], add=True)
            if prefetch:
                if first:
                    fire(i + 2, b2)
                else:
                    drain_scatter(b2)
                    fire(i + 2, b2)

        def step(j, carry):
            i0 = 3 * j

            @pl.when(j == 0)
            def _():
                for b in range(3):
                    handle(b, b, (b + 2) % 3, True, b == 0)

            @pl.when(j > 0)
            def _():
                for b in range(3):
                    i = i0 + b
                    handle(i, b, (b + 2) % 3, True, False)
            return carry

        ngroup = NCHUNK // 3
        lax.fori_loop(0, ngroup, step, 0)
        for i, b in ((NCHUNK - 2, 0), (NCHUNK - 1, 1)):
            wait_gather(i, b)
            pltpu.async_copy(rbufs[b], acc.at[cidxs[b]], ssems[b], add=True)
        for b in range(3):
            drain_scatter(b)

        plsc.subcore_barrier()
        pltpu.sync_copy(
            acc.at[pl.ds(s * RPT, RPT)],
            out_hbm.at[pl.ds(c * NP + s * RPT, RPT)],
        )

    f = pl.kernel(
        body,
        out_type=jax.ShapeDtypeStruct((NC * NP, H), jnp.float32),
        mesh=_mesh(),
        scratch_types=[
            pltpu.VMEM((EPW,), jnp.int32),
            pltpu.VMEM((CH,), jnp.int32),
            pltpu.VMEM((CH,), jnp.int32),
            pltpu.VMEM((CH,), jnp.int32),
            pltpu.VMEM((CH, H), jnp.float32),
            pltpu.VMEM((CH, H), jnp.float32),
            pltpu.VMEM((CH, H), jnp.float32),
            pltpu.VMEM_SHARED((NP, H), jnp.float32),
            pltpu.SemaphoreType.DMA,
            pltpu.SemaphoreType.DMA,
            pltpu.SemaphoreType.DMA,
            pltpu.SemaphoreType.DMA,
            pltpu.SemaphoreType.DMA,
            pltpu.SemaphoreType.DMA,
        ],
    )
    return f(row1, col1, g, zeros_nh)


def _mm_call(x, w, dp):
    def body(x_ref, w_ref, dp_ref, g_ref, dinv_ref):
        deg = dp_ref[0:N] + dp_ref[N : 2 * N] + 1.0
        dinv = lax.rsqrt(deg)
        h = jnp.dot(x_ref[...], w_ref[...], preferred_element_type=jnp.float32)
        g_ref[...] = h * dinv
        dinv_ref[...] = dinv

    return pl.pallas_call(
        body,
        out_shape=(
            jax.ShapeDtypeStruct((N, H), jnp.float32),
            jax.ShapeDtypeStruct((N, 1), jnp.float32),
        ),
    )(x, w, dp)


def _fin_call(p, g, dinv, gcn_b, bn_gamma, bn_beta, proj_W, proj_b, item_table):
    def body(p_ref, g_ref, dinv_ref, b_ref, gam_ref, bet_ref, pw_ref, pb_ref, it_ref,
             scores_ref, rsu_ref):
        s_sum = p_ref[0:N] + p_ref[NP : NP + N]
        agg = dinv_ref[...] * (s_sum + g_ref[...]) + b_ref[...]
        mean = jnp.sum(agg, axis=0, keepdims=True) * (1.0 / N)
        cen = agg - mean
        var = jnp.sum(cen * cen, axis=0, keepdims=True) * (1.0 / N)
        y0 = cen[0:1, :] * lax.rsqrt(var + 1e-5) * gam_ref[...] + bet_ref[...]
        y0 = jnp.maximum(y0, 0.0)
        rsu = (
            lax.dot_general(y0, pw_ref[...], (((1,), (1,)), ((), ())),
                            preferred_element_type=jnp.float32)
            + pb_ref[...]
        )
        rsu_ref[...] = rsu
        scores_ref[...] = lax.dot_general(rsu, it_ref[...], (((1,), (1,)), ((), ())),
                                          preferred_element_type=jnp.float32)

    return pl.pallas_call(
        body,
        out_shape=(
            jax.ShapeDtypeStruct((1, NUM_ITEMS), jnp.float32),
            jax.ShapeDtypeStruct((1, D), jnp.float32),
        ),
    )(p, g, dinv, gcn_b, bn_gamma, bn_beta, proj_W, proj_b, item_table)


def kernel(node_feature, edge_index, items_ready_to_cache, gcn_W, gcn_b,
           bn_gamma, bn_beta, proj_W, proj_b, item_table):
    row1 = edge_index[0]
    col1 = edge_index[1]
    col2 = col1.reshape(NW, NCHUNK, CH)
    zeros_n = jnp.zeros((N,), jnp.float32)
    zeros_nh = jnp.zeros((RPT, H), jnp.float32)
    dp = _deg_call(col2, zeros_n)
    g, dinv = _mm_call(node_feature, gcn_W, dp.reshape(2 * N, 1))
    p = _scatter_call(row1, col1, g, zeros_nh)
    scores2, rsu = _fin_call(
        p, g, dinv,
        gcn_b.reshape(1, H), bn_gamma.reshape(1, H), bn_beta.reshape(1, H),
        proj_W, proj_b.reshape(1, D), item_table,
    )
    return scores2.reshape(NUM_ITEMS), rsu


# speedup vs baseline: 47.0738x; 1.0655x over previous
import jax
import jax.numpy as jnp
from jax import lax
from jax.experimental import pallas as pl
from jax.experimental.pallas import tpu as pltpu
from jax.experimental.pallas import tpu_sc as plsc

N = 10000
D = 128
H = 128
E = 320000
NUM_ITEMS = 128

NC = 2
NS = 16
NW = NC * NS
EPW = E // NW
CH = 80
NCHUNK = EPW // CH
NP = 10240
RPT = NP // NS


def _mesh():
    return plsc.VectorSubcoreMesh(core_axis_name="c", subcore_axis_name="s")


def _deg_call(col2, zeros_n):
    WIN = 8

    def body(col_hbm, zeros_hbm, out_hbm, cidx_all, ones, dbuf, dacc, sem):
        c = lax.axis_index("c")
        s = lax.axis_index("s")
        wid = c * NS + s
        for j in range(CH // 16):
            ones[pl.ds(j * 16, 16)] = jnp.ones((16,), jnp.float32)
        pltpu.sync_copy(col_hbm.at[wid], cidx_all)

        @pl.when(s == 0)
        def _():
            pltpu.sync_copy(zeros_hbm, dbuf)
            pltpu.sync_copy(dbuf, dacc)

        plsc.subcore_barrier()

        def fire(i):
            pltpu.async_copy(ones, dacc.at[cidx_all.at[i]], sem, add=True)

        def drain_one():
            pltpu.make_async_copy(ones, dacc.at[cidx_all.at[0]], sem).wait()

        for i in range(WIN):
            fire(i)

        def step(i, carry):
            drain_one()
            fire(i + WIN)
            return carry

        lax.fori_loop(0, NCHUNK - WIN, step, 0)
        for _ in range(WIN):
            drain_one()
        plsc.subcore_barrier()

        @pl.when(s == 0)
        def _():
            pltpu.sync_copy(dacc, dbuf)
            pltpu.sync_copy(dbuf, out_hbm.at[pl.ds(c * N, N)])

    f = pl.kernel(
        body,
        out_type=jax.ShapeDtypeStruct((NC * N,), jnp.float32),
        mesh=_mesh(),
        scratch_types=[
            pltpu.VMEM((NCHUNK, CH), jnp.int32),
            pltpu.VMEM((CH,), jnp.float32),
            pltpu.VMEM((N,), jnp.float32),
            pltpu.VMEM_SHARED((N,), jnp.float32),
            pltpu.SemaphoreType.DMA,
        ],
    )
    return f(col2, zeros_n)


def _scatter_call(row1, col1, g, zeros_nh):
    def body(row_hbm, col_hbm, g_hbm, zeros_hbm, out_hbm,
             ridx_all, cidx0, cidx1, cidx2, rbuf0, rbuf1, rbuf2, acc,
             gsem0, gsem1, gsem2, ssem0, ssem1, ssem2):
        c = lax.axis_index("c")
        s = lax.axis_index("s")
        wid = c * NS + s
        pltpu.sync_copy(zeros_hbm, acc.at[pl.ds(s * RPT, RPT)])
        pltpu.sync_copy(row_hbm.at[pl.ds(wid * EPW, EPW)], ridx_all)
        plsc.subcore_barrier()

        rbufs = (rbuf0, rbuf1, rbuf2)
        cidxs = (cidx0, cidx1, cidx2)
        gsems = (gsem0, gsem1, gsem2)
        ssems = (ssem0, ssem1, ssem2)

        def gsrc(i):
            return g_hbm.at[ridx_all.at[pl.ds(i * CH, CH)]]

        def csrc(i):
            return col_hbm.at[pl.ds(wid * EPW + i * CH, CH)]

        def fire(i, b):
            pltpu.async_copy(csrc(i), cidxs[b], gsems[b])
            pltpu.async_copy(gsrc(i), rbufs[b], gsems[b])

        def wait_gather(i, b):
            pltpu.make_async_copy(csrc(i), cidxs[b], gsems[b]).wait()
            pltpu.make_async_copy(gsrc(i), rbufs[b], gsems[b]).wait()

        def drain_scatter(b):
            pltpu.make_async_copy(g_hbm.at[pl.ds(0, CH)], rbufs[b], ssems[b]).wait()

        fire(0, 0)
        fire(1, 1)

        def handle(i, b, b2, prefetch, first):
            wait_gather(i, b)
            pltpu.async_copy(rbufs[b], acc.at[cidxs[b]], ssems[b], add=True)
            if prefetch:
                if first:
                    fire(i + 2, b2)
                else:
                    drain_scatter(b2)
                    fire(i + 2, b2)

        def step(j, carry):
            i0 = 3 * j

            @pl.when(j == 0)
            def _():
                for b in range(3):
                    handle(b, b, (b + 2) % 3, True, b == 0)

            @pl.when(j > 0)
            def _():
                for b in range(3):
                    i = i0 + b
                    handle(i, b, (b + 2) % 3, True, False)
            return carry

        ngroup = NCHUNK // 3
        lax.fori_loop(0, ngroup, step, 0)
        for i, b in ((NCHUNK - 2, 0), (NCHUNK - 1, 1)):
            wait_gather(i, b)
            pltpu.async_copy(rbufs[b], acc.at[cidxs[b]], ssems[b], add=True)
        for b in range(3):
            drain_scatter(b)

        plsc.subcore_barrier()
        pltpu.sync_copy(
            acc.at[pl.ds(s * RPT, RPT)],
            out_hbm.at[pl.ds(c * NP + s * RPT, RPT)],
        )

    f = pl.kernel(
        body,
        out_type=jax.ShapeDtypeStruct((NC * NP, H), jnp.float32),
        mesh=_mesh(),
        scratch_types=[
            pltpu.VMEM((EPW,), jnp.int32),
            pltpu.VMEM((CH,), jnp.int32),
            pltpu.VMEM((CH,), jnp.int32),
            pltpu.VMEM((CH,), jnp.int32),
            pltpu.VMEM((CH, H), jnp.float32),
            pltpu.VMEM((CH, H), jnp.float32),
            pltpu.VMEM((CH, H), jnp.float32),
            pltpu.VMEM_SHARED((NP, H), jnp.float32),
            pltpu.SemaphoreType.DMA,
            pltpu.SemaphoreType.DMA,
            pltpu.SemaphoreType.DMA,
            pltpu.SemaphoreType.DMA,
            pltpu.SemaphoreType.DMA,
            pltpu.SemaphoreType.DMA,
        ],
    )
    return f(row1, col1, g, zeros_nh)


def _mm_call(x, w, dp2):
    def body(x_ref, w_ref, dp_ref, g_ref, dinv_ref):
        deg = dp_ref[0:1, :] + dp_ref[1:2, :] + 1.0
        dinv_row = lax.rsqrt(deg)
        dinv_col = lax.transpose(dinv_row, (1, 0))
        h = jnp.dot(x_ref[...], w_ref[...], preferred_element_type=jnp.float32)
        g_ref[...] = h * dinv_col
        dinv_ref[...] = dinv_row

    return pl.pallas_call(
        body,
        out_shape=(
            jax.ShapeDtypeStruct((N, H), jnp.float32),
            jax.ShapeDtypeStruct((1, N), jnp.float32),
        ),
    )(x, w, dp2)


def _fin_call(p, g, dinv, gcn_b, bn_gamma, bn_beta, proj_W, proj_b, item_table):
    def body(p_ref, g_ref, dinv_ref, b_ref, gam_ref, bet_ref, pw_ref, pb_ref, it_ref,
             scores_ref, rsu_ref):
        s_sum = p_ref[0:N] + p_ref[NP : NP + N]
        dinv_col = lax.transpose(dinv_ref[...], (1, 0))
        agg = dinv_col * (s_sum + g_ref[...]) + b_ref[...]
        mean = jnp.sum(agg, axis=0, keepdims=True) * (1.0 / N)
        cen = agg - mean
        var = jnp.sum(cen * cen, axis=0, keepdims=True) * (1.0 / N)
        y0 = cen[0:1, :] * lax.rsqrt(var + 1e-5) * gam_ref[...] + bet_ref[...]
        y0 = jnp.maximum(y0, 0.0)
        rsu = (
            lax.dot_general(y0, pw_ref[...], (((1,), (1,)), ((), ())),
                            preferred_element_type=jnp.float32)
            + pb_ref[...]
        )
        rsu_ref[...] = rsu
        scores_ref[...] = lax.dot_general(rsu, it_ref[...], (((1,), (1,)), ((), ())),
                                          preferred_element_type=jnp.float32)

    return pl.pallas_call(
        body,
        out_shape=(
            jax.ShapeDtypeStruct((1, NUM_ITEMS), jnp.float32),
            jax.ShapeDtypeStruct((1, D), jnp.float32),
        ),
    )(p, g, dinv, gcn_b, bn_gamma, bn_beta, proj_W, proj_b, item_table)


def kernel(node_feature, edge_index, items_ready_to_cache, gcn_W, gcn_b,
           bn_gamma, bn_beta, proj_W, proj_b, item_table):
    row1 = edge_index[0]
    col1 = edge_index[1]
    col2 = col1.reshape(NW, NCHUNK, CH)
    zeros_n = jnp.zeros((N,), jnp.float32)
    zeros_nh = jnp.zeros((RPT, H), jnp.float32)
    dp = _deg_call(col2, zeros_n)
    g, dinv = _mm_call(node_feature, gcn_W, dp.reshape(2, N))
    p = _scatter_call(row1, col1, g, zeros_nh)
    scores2, rsu = _fin_call(
        p, g, dinv,
        gcn_b.reshape(1, H), bn_gamma.reshape(1, H), bn_beta.reshape(1, H),
        proj_W, proj_b.reshape(1, D), item_table,
    )
    return scores2.reshape(NUM_ITEMS), rsu


# speedup vs baseline: 47.8551x; 1.0166x over previous
import jax
import jax.numpy as jnp
from jax import lax
from jax.experimental import pallas as pl
from jax.experimental.pallas import tpu as pltpu
from jax.experimental.pallas import tpu_sc as plsc

N = 10000
D = 128
H = 128
E = 320000
NUM_ITEMS = 128

NC = 2
NS = 16
NW = NC * NS
EPW = E // NW
CH = 80
NCHUNK = EPW // CH
NP = 10240
RPT = NP // NS


def _mesh():
    return plsc.VectorSubcoreMesh(core_axis_name="c", subcore_axis_name="s")


def _deg_call(col2, zeros_n):
    WIN = 8

    def body(col_hbm, zeros_hbm, out_hbm, cidx_all, ones, dbuf, dacc, sem):
        c = lax.axis_index("c")
        s = lax.axis_index("s")
        wid = c * NS + s
        for j in range(CH // 16):
            ones[pl.ds(j * 16, 16)] = jnp.ones((16,), jnp.float32)
        pltpu.sync_copy(col_hbm.at[wid], cidx_all)

        @pl.when(s == 0)
        def _():
            pltpu.sync_copy(zeros_hbm, dbuf)
            pltpu.sync_copy(dbuf, dacc)

        plsc.subcore_barrier()

        def fire(i):
            pltpu.async_copy(ones, dacc.at[cidx_all.at[i]], sem, add=True)

        def drain_one():
            pltpu.make_async_copy(ones, dacc.at[cidx_all.at[0]], sem).wait()

        for i in range(WIN):
            fire(i)

        def step(i, carry):
            drain_one()
            fire(i + WIN)
            return carry

        lax.fori_loop(0, NCHUNK - WIN, step, 0)
        for _ in range(WIN):
            drain_one()
        plsc.subcore_barrier()

        @pl.when(s == 0)
        def _():
            pltpu.sync_copy(dacc, dbuf)
            pltpu.sync_copy(dbuf, out_hbm.at[pl.ds(c * N, N)])

    f = pl.kernel(
        body,
        out_type=jax.ShapeDtypeStruct((NC * N,), jnp.float32),
        mesh=_mesh(),
        scratch_types=[
            pltpu.VMEM((NCHUNK, CH), jnp.int32),
            pltpu.VMEM((CH,), jnp.float32),
            pltpu.VMEM((N,), jnp.float32),
            pltpu.VMEM_SHARED((N,), jnp.float32),
            pltpu.SemaphoreType.DMA,
        ],
    )
    return f(col2, zeros_n)


def _scatter_call(row1, col1, g, zeros_nh):
    def body(row_hbm, col_hbm, g_hbm, zeros_hbm, out_hbm,
             ridx_all, cidx0, cidx1, cidx2, rbuf0, rbuf1, rbuf2, acc,
             gsem0, gsem1, gsem2, ssem0, ssem1, ssem2):
        c = lax.axis_index("c")
        s = lax.axis_index("s")
        wid = c * NS + s

        @pl.when(c == 0)
        def _():
            pltpu.sync_copy(zeros_hbm, acc.at[pl.ds(s * RPT, RPT)])

        @pl.when(jnp.logical_and(c == 1, s < NS - 1))
        def _():
            pltpu.sync_copy(g_hbm.at[pl.ds(s * RPT, RPT)],
                            acc.at[pl.ds(s * RPT, RPT)])

        @pl.when(jnp.logical_and(c == 1, s == NS - 1))
        def _():
            last = N - (NS - 1) * RPT
            pltpu.sync_copy(g_hbm.at[pl.ds((NS - 1) * RPT, last)],
                            acc.at[pl.ds((NS - 1) * RPT, last)])
            pltpu.sync_copy(zeros_hbm.at[pl.ds(0, NP - N)],
                            acc.at[pl.ds(N, NP - N)])

        pltpu.sync_copy(row_hbm.at[pl.ds(wid * EPW, EPW)], ridx_all)
        plsc.subcore_barrier()

        rbufs = (rbuf0, rbuf1, rbuf2)
        cidxs = (cidx0, cidx1, cidx2)
        gsems = (gsem0, gsem1, gsem2)
        ssems = (ssem0, ssem1, ssem2)

        def gsrc(i):
            return g_hbm.at[ridx_all.at[pl.ds(i * CH, CH)]]

        def csrc(i):
            return col_hbm.at[pl.ds(wid * EPW + i * CH, CH)]

        def fire(i, b):
            pltpu.async_copy(csrc(i), cidxs[b], gsems[b])
            pltpu.async_copy(gsrc(i), rbufs[b], gsems[b])

        def wait_gather(i, b):
            pltpu.make_async_copy(csrc(i), cidxs[b], gsems[b]).wait()
            pltpu.make_async_copy(gsrc(i), rbufs[b], gsems[b]).wait()

        def drain_scatter(b):
            pltpu.make_async_copy(g_hbm.at[pl.ds(0, CH)], rbufs[b], ssems[b]).wait()

        fire(0, 0)
        fire(1, 1)

        def handle(i, b, b2, prefetch, first):
            wait_gather(i, b)
            pltpu.async_copy(rbufs[b], acc.at[cidxs[b]], ssems[b], add=True)
            if prefetch:
                if first:
                    fire(i + 2, b2)
                else:
                    drain_scatter(b2)
                    fire(i + 2, b2)

        def step(j, carry):
            i0 = 3 * j

            @pl.when(j == 0)
            def _():
                for b in range(3):
                    handle(b, b, (b + 2) % 3, True, b == 0)

            @pl.when(j > 0)
            def _():
                for b in range(3):
                    i = i0 + b
                    handle(i, b, (b + 2) % 3, True, False)
            return carry

        ngroup = NCHUNK // 3
        lax.fori_loop(0, ngroup, step, 0)
        for i, b in ((NCHUNK - 2, 0), (NCHUNK - 1, 1)):
            wait_gather(i, b)
            pltpu.async_copy(rbufs[b], acc.at[cidxs[b]], ssems[b], add=True)
        for b in range(3):
            drain_scatter(b)

        plsc.subcore_barrier()
        pltpu.sync_copy(
            acc.at[pl.ds(s * RPT, RPT)],
            out_hbm.at[pl.ds(c * NP + s * RPT, RPT)],
        )

    f = pl.kernel(
        body,
        out_type=jax.ShapeDtypeStruct((NC * NP, H), jnp.float32),
        mesh=_mesh(),
        scratch_types=[
            pltpu.VMEM((EPW,), jnp.int32),
            pltpu.VMEM((CH,), jnp.int32),
            pltpu.VMEM((CH,), jnp.int32),
            pltpu.VMEM((CH,), jnp.int32),
            pltpu.VMEM((CH, H), jnp.float32),
            pltpu.VMEM((CH, H), jnp.float32),
            pltpu.VMEM((CH, H), jnp.float32),
            pltpu.VMEM_SHARED((NP, H), jnp.float32),
            pltpu.SemaphoreType.DMA,
            pltpu.SemaphoreType.DMA,
            pltpu.SemaphoreType.DMA,
            pltpu.SemaphoreType.DMA,
            pltpu.SemaphoreType.DMA,
            pltpu.SemaphoreType.DMA,
        ],
    )
    return f(row1, col1, g, zeros_nh)


def _mm_call(x, w, dp2):
    def body(x_ref, w_ref, dp_ref, g_ref, dinv_ref):
        deg = dp_ref[0:1, :] + dp_ref[1:2, :] + 1.0
        dinv_row = lax.rsqrt(deg)
        dinv_col = lax.transpose(dinv_row, (1, 0))
        h = jnp.dot(x_ref[...], w_ref[...], preferred_element_type=jnp.float32)
        g_ref[...] = h * dinv_col
        dinv_ref[...] = dinv_row

    return pl.pallas_call(
        body,
        out_shape=(
            jax.ShapeDtypeStruct((N, H), jnp.float32),
            jax.ShapeDtypeStruct((1, N), jnp.float32),
        ),
    )(x, w, dp2)


def _fin_call(p, dinv, gcn_b, bn_gamma, bn_beta, proj_W, proj_b, item_table):
    def body(p_ref, dinv_ref, b_ref, gam_ref, bet_ref, pw_ref, pb_ref, it_ref,
             scores_ref, rsu_ref):
        s_sum = p_ref[0:N] + p_ref[NP : NP + N]
        dinv_col = lax.transpose(dinv_ref[...], (1, 0))
        agg = dinv_col * s_sum + b_ref[...]
        mean = jnp.sum(agg, axis=0, keepdims=True) * (1.0 / N)
        cen = agg - mean
        var = jnp.sum(cen * cen, axis=0, keepdims=True) * (1.0 / N)
        y0 = cen[0:1, :] * lax.rsqrt(var + 1e-5) * gam_ref[...] + bet_ref[...]
        y0 = jnp.maximum(y0, 0.0)
        rsu = (
            lax.dot_general(y0, pw_ref[...], (((1,), (1,)), ((), ())),
                            preferred_element_type=jnp.float32)
            + pb_ref[...]
        )
        rsu_ref[...] = rsu
        scores_ref[...] = lax.dot_general(rsu, it_ref[...], (((1,), (1,)), ((), ())),
                                          preferred_element_type=jnp.float32)

    return pl.pallas_call(
        body,
        out_shape=(
            jax.ShapeDtypeStruct((1, NUM_ITEMS), jnp.float32),
            jax.ShapeDtypeStruct((1, D), jnp.float32),
        ),
    )(p, dinv, gcn_b, bn_gamma, bn_beta, proj_W, proj_b, item_table)


def kernel(node_feature, edge_index, items_ready_to_cache, gcn_W, gcn_b,
           bn_gamma, bn_beta, proj_W, proj_b, item_table):
    row1 = edge_index[0]
    col1 = edge_index[1]
    col2 = col1.reshape(NW, NCHUNK, CH)
    zeros_n = jnp.zeros((N,), jnp.float32)
    zeros_nh = jnp.zeros((RPT, H), jnp.float32)
    dp = _deg_call(col2, zeros_n)
    g, dinv = _mm_call(node_feature, gcn_W, dp.reshape(2, N))
    p = _scatter_call(row1, col1, g, zeros_nh)
    scores2, rsu = _fin_call(
        p, dinv,
        gcn_b.reshape(1, H), bn_gamma.reshape(1, H), bn_beta.reshape(1, H),
        proj_W, proj_b.reshape(1, D), item_table,
    )
    return scores2.reshape(NUM_ITEMS), rsu


# speedup vs baseline: 48.0929x; 1.0050x over previous
import jax
import jax.numpy as jnp
from jax import lax
from jax.experimental import pallas as pl
from jax.experimental.pallas import tpu as pltpu
from jax.experimental.pallas import tpu_sc as plsc

N = 10000
D = 128
H = 128
E = 320000
NUM_ITEMS = 128

NC = 2
NS = 16
NW = NC * NS
EPW = E // NW
CH = 80
NCHUNK = EPW // CH
NP = 10240
RPT = NP // NS


def _mesh():
    return plsc.VectorSubcoreMesh(core_axis_name="c", subcore_axis_name="s")


def _deg_call(col2, zeros_n):
    WIN = 8

    def body(col_hbm, zeros_hbm, out_hbm, cidx_all, ones, dbuf, dacc, sem):
        c = lax.axis_index("c")
        s = lax.axis_index("s")
        wid = c * NS + s
        for j in range(CH // 16):
            ones[pl.ds(j * 16, 16)] = jnp.ones((16,), jnp.float32)
        pltpu.sync_copy(col_hbm.at[wid], cidx_all)

        @pl.when(s == 0)
        def _():
            pltpu.sync_copy(zeros_hbm, dbuf)
            pltpu.sync_copy(dbuf, dacc)

        plsc.subcore_barrier()

        def fire(i):
            pltpu.async_copy(ones, dacc.at[cidx_all.at[i]], sem, add=True)

        def drain_one():
            pltpu.make_async_copy(ones, dacc.at[cidx_all.at[0]], sem).wait()

        for i in range(WIN):
            fire(i)

        def step(i, carry):
            drain_one()
            fire(i + WIN)
            return carry

        lax.fori_loop(0, NCHUNK - WIN, step, 0)
        for _ in range(WIN):
            drain_one()
        plsc.subcore_barrier()

        @pl.when(s == 0)
        def _():
            pltpu.sync_copy(dacc, dbuf)
            pltpu.sync_copy(dbuf, out_hbm.at[pl.ds(c * N, N)])

    f = pl.kernel(
        body,
        out_type=jax.ShapeDtypeStruct((NC * N,), jnp.float32),
        mesh=_mesh(),
        scratch_types=[
            pltpu.VMEM((NCHUNK, CH), jnp.int32),
            pltpu.VMEM((CH,), jnp.float32),
            pltpu.VMEM((N,), jnp.float32),
            pltpu.VMEM_SHARED((N,), jnp.float32),
            pltpu.SemaphoreType.DMA,
        ],
    )
    return f(col2, zeros_n)


def _scatter_call(row1, col1, g, zeros_nh):
    def body(row_hbm, col_hbm, g_hbm, zeros_hbm, out_hbm,
             ridx_all, cidx0, cidx1, cidx2, rbuf0, rbuf1, rbuf2, acc,
             gsem0, gsem1, gsem2, ssem0, ssem1, ssem2):
        c = lax.axis_index("c")
        s = lax.axis_index("s")
        wid = c * NS + s

        @pl.when(c == 0)
        def _():
            pltpu.sync_copy(zeros_hbm, acc.at[pl.ds(s * RPT, RPT)])

        @pl.when(jnp.logical_and(c == 1, s < NS - 1))
        def _():
            pltpu.sync_copy(g_hbm.at[pl.ds(s * RPT, RPT)],
                            acc.at[pl.ds(s * RPT, RPT)])

        @pl.when(jnp.logical_and(c == 1, s == NS - 1))
        def _():
            last = N - (NS - 1) * RPT
            pltpu.sync_copy(g_hbm.at[pl.ds((NS - 1) * RPT, last)],
                            acc.at[pl.ds((NS - 1) * RPT, last)])
            pltpu.sync_copy(zeros_hbm.at[pl.ds(0, NP - N)],
                            acc.at[pl.ds(N, NP - N)])

        pltpu.sync_copy(row_hbm.at[pl.ds(wid * EPW, EPW)], ridx_all)

        rbufs = (rbuf0, rbuf1, rbuf2)
        cidxs = (cidx0, cidx1, cidx2)
        gsems = (gsem0, gsem1, gsem2)
        ssems = (ssem0, ssem1, ssem2)

        def gsrc(i):
            return g_hbm.at[ridx_all.at[pl.ds(i * CH, CH)]]

        def csrc(i):
            return col_hbm.at[pl.ds(wid * EPW + i * CH, CH)]

        def fire(i, b):
            pltpu.async_copy(csrc(i), cidxs[b], gsems[b])
            pltpu.async_copy(gsrc(i), rbufs[b], gsems[b])

        def wait_gather(i, b):
            pltpu.make_async_copy(csrc(i), cidxs[b], gsems[b]).wait()
            pltpu.make_async_copy(gsrc(i), rbufs[b], gsems[b]).wait()

        def drain_scatter(b):
            pltpu.make_async_copy(g_hbm.at[pl.ds(0, CH)], rbufs[b], ssems[b]).wait()

        fire(0, 0)
        fire(1, 1)
        plsc.subcore_barrier()

        def handle(i, b, b2, prefetch, first):
            wait_gather(i, b)
            pltpu.async_copy(rbufs[b], acc.at[cidxs[b]], ssems[b], add=True)
            if prefetch:
                if first:
                    fire(i + 2, b2)
                else:
                    drain_scatter(b2)
                    fire(i + 2, b2)

        def step(j, carry):
            i0 = 3 * j

            @pl.when(j == 0)
            def _():
                for b in range(3):
                    handle(b, b, (b + 2) % 3, True, b == 0)

            @pl.when(j > 0)
            def _():
                for b in range(3):
                    i = i0 + b
                    handle(i, b, (b + 2) % 3, True, False)
            return carry

        ngroup = NCHUNK // 3
        lax.fori_loop(0, ngroup, step, 0)
        for i, b in ((NCHUNK - 2, 0), (NCHUNK - 1, 1)):
            wait_gather(i, b)
            pltpu.async_copy(rbufs[b], acc.at[cidxs[b]], ssems[b], add=True)
        for b in range(3):
            drain_scatter(b)

        plsc.subcore_barrier()
        pltpu.sync_copy(
            acc.at[pl.ds(s * RPT, RPT)],
            out_hbm.at[pl.ds(c * NP + s * RPT, RPT)],
        )

    f = pl.kernel(
        body,
        out_type=jax.ShapeDtypeStruct((NC * NP, H), jnp.float32),
        mesh=_mesh(),
        scratch_types=[
            pltpu.VMEM((EPW,), jnp.int32),
            pltpu.VMEM((CH,), jnp.int32),
            pltpu.VMEM((CH,), jnp.int32),
            pltpu.VMEM((CH,), jnp.int32),
            pltpu.VMEM((CH, H), jnp.float32),
            pltpu.VMEM((CH, H), jnp.float32),
            pltpu.VMEM((CH, H), jnp.float32),
            pltpu.VMEM_SHARED((NP, H), jnp.float32),
            pltpu.SemaphoreType.DMA,
            pltpu.SemaphoreType.DMA,
            pltpu.SemaphoreType.DMA,
            pltpu.SemaphoreType.DMA,
            pltpu.SemaphoreType.DMA,
            pltpu.SemaphoreType.DMA,
        ],
    )
    return f(row1, col1, g, zeros_nh)


def _mm_call(x, w, dp2):
    def body(x_ref, w_ref, dp_ref, g_ref, dinv_ref):
        deg = dp_ref[0:1, :] + dp_ref[1:2, :] + 1.0
        dinv_row = lax.rsqrt(deg)
        dinv_col = lax.transpose(dinv_row, (1, 0))
        h = jnp.dot(x_ref[...], w_ref[...], preferred_element_type=jnp.float32)
        g_ref[...] = h * dinv_col
        dinv_ref[...] = dinv_row

    return pl.pallas_call(
        body,
        out_shape=(
            jax.ShapeDtypeStruct((N, H), jnp.float32),
            jax.ShapeDtypeStruct((1, N), jnp.float32),
        ),
    )(x, w, dp2)


def _fin_call(p, dinv, gcn_b, bn_gamma, bn_beta, proj_W, proj_b, item_table):
    def body(p_ref, dinv_ref, b_ref, gam_ref, bet_ref, pw_ref, pb_ref, it_ref,
             scores_ref, rsu_ref):
        s_sum = p_ref[0:N] + p_ref[NP : NP + N]
        dinv_col = lax.transpose(dinv_ref[...], (1, 0))
        agg = dinv_col * s_sum + b_ref[...]
        mean = jnp.sum(agg, axis=0, keepdims=True) * (1.0 / N)
        cen = agg - mean
        var = jnp.sum(cen * cen, axis=0, keepdims=True) * (1.0 / N)
        y0 = cen[0:1, :] * lax.rsqrt(var + 1e-5) * gam_ref[...] + bet_ref[...]
        y0 = jnp.maximum(y0, 0.0)
        rsu = (
            lax.dot_general(y0, pw_ref[...], (((1,), (1,)), ((), ())),
                            preferred_element_type=jnp.float32)
            + pb_ref[...]
        )
        rsu_ref[...] = rsu
        scores_ref[...] = lax.dot_general(rsu, it_ref[...], (((1,), (1,)), ((), ())),
                                          preferred_element_type=jnp.float32)

    return pl.pallas_call(
        body,
        out_shape=(
            jax.ShapeDtypeStruct((1, NUM_ITEMS), jnp.float32),
            jax.ShapeDtypeStruct((1, D), jnp.float32),
        ),
    )(p, dinv, gcn_b, bn_gamma, bn_beta, proj_W, proj_b, item_table)


def kernel(node_feature, edge_index, items_ready_to_cache, gcn_W, gcn_b,
           bn_gamma, bn_beta, proj_W, proj_b, item_table):
    row1 = edge_index[0]
    col1 = edge_index[1]
    col2 = col1.reshape(NW, NCHUNK, CH)
    zeros_n = jnp.zeros((N,), jnp.float32)
    zeros_nh = jnp.zeros((RPT, H), jnp.float32)
    dp = _deg_call(col2, zeros_n)
    g, dinv = _mm_call(node_feature, gcn_W, dp.reshape(2, N))
    p = _scatter_call(row1, col1, g, zeros_nh)
    scores2, rsu = _fin_call(
        p, dinv,
        gcn_b.reshape(1, H), bn_gamma.reshape(1, H), bn_beta.reshape(1, H),
        proj_W, proj_b.reshape(1, D), item_table,
    )
    return scores2.reshape(NUM_ITEMS), rsu
